# baseline (speedup 1.0000x reference)
import functools

import jax
import jax.numpy as jnp
from jax import lax
from jax.experimental import pallas as pl
from jax.experimental.pallas import tpu as pltpu
from jax.experimental.pallas import tpu_sc as plsc

N = 10000
F_IN = 128
F_H = 32
N_GRAPHS = 64

NSUB = 16
LANES = 16
NPAD = 10240
CHUNK = NPAD // NSUB
VREGS = CHUNK // LANES
BINS = 80
PAD_ID = 64
FORCE_ID = 66


def _tc_body(x_ref, wz0_ref, wz1_ref, bz_ref, wh0_ref, wh1_ref, bh_ref,
             wl_ref, g_ref):
    x = x_ref[...]
    az = wz0_ref[0:F_IN, :] + wz1_ref[0:F_IN, :]
    ah = wh0_ref[0:F_IN, :] + wh1_ref[0:F_IN, :]
    pz = jnp.dot(x, az, preferred_element_type=jnp.float32) + bz_ref[...]
    ph = jnp.dot(x, ah, preferred_element_type=jnp.float32) + bh_ref[...]
    z = jax.nn.sigmoid(pz)
    t = jnp.tanh(ph)
    hr = jnp.maximum((1.0 - z) * t, 0.0)
    g_ref[...] = jnp.dot(hr, wl_ref[...], preferred_element_type=jnp.float32)


def _sc_body(g_hbm, b_hbm, bp_hbm, bl_hbm, out_hbm,
             g_v, b_v, bp_v, acc_s, acc_c,
             stage_s, stage_c, comb_v, out_v, bl_v):
    wid = lax.axis_index("s")
    base = wid * CHUNK
    pltpu.sync_copy(g_hbm.at[pl.ds(base, CHUNK)], g_v)
    pltpu.sync_copy(b_hbm.at[pl.ds(base, CHUNK)], b_v)
    pltpu.sync_copy(bp_hbm.at[pl.ds(base, CHUNK)], bp_v)

    zero16 = jnp.zeros((LANES,), jnp.float32)
    for k in range(BINS // LANES):
        acc_s[pl.ds(k * LANES, LANES)] = zero16
        acc_c[pl.ds(k * LANES, LANES)] = zero16

    def step(j, carry):
        v = g_v[pl.ds(j * LANES, LANES)]
        bb = b_v[pl.ds(j * LANES, LANES)]
        bp = bp_v[pl.ds(j * LANES, LANES)]
        cs = plsc.cumsum(v)
        excl = carry + cs - v
        mask = bb != bp
        plsc.addupdate_scatter(acc_s, [bb], -excl, mask=mask)
        plsc.addupdate_scatter(acc_s, [bp], excl, mask=mask)
        lpos = (j * LANES + lax.iota(jnp.int32, LANES)).astype(jnp.float32)
        plsc.addupdate_scatter(acc_c, [bb], -lpos, mask=mask)
        plsc.addupdate_scatter(acc_c, [bp], lpos, mask=mask)
        return carry + jnp.broadcast_to(jnp.sum(v), (LANES,))
    carry = lax.fori_loop(0, VREGS, step, zero16)

    bb_last = b_v[pl.ds((VREGS - 1) * LANES, LANES)]
    last_lane = lax.iota(jnp.int32, LANES) == (LANES - 1)
    plsc.addupdate_scatter(acc_s, [bb_last], carry, mask=last_lane)
    plsc.addupdate_scatter(acc_c, [bb_last],
                           jnp.full((LANES,), float(CHUNK), jnp.float32),
                           mask=last_lane)

    pltpu.sync_copy(acc_s, stage_s.at[pl.ds(wid * BINS, BINS)])
    pltpu.sync_copy(acc_c, stage_c.at[pl.ds(wid * BINS, BINS)])
    plsc.subcore_barrier()

    @pl.when(wid == 0)
    def _():
        pltpu.sync_copy(bl_hbm, bl_v)
        pltpu.sync_copy(stage_s, comb_v.at[pl.ds(0, NSUB * BINS)])
        pltpu.sync_copy(stage_c, comb_v.at[pl.ds(NSUB * BINS, NSUB * BINS)])
        blv = bl_v[...]
        for k in range(N_GRAPHS // LANES):
            s_tot = zero16
            c_tot = zero16
            for r in range(NSUB):
                s_tot = s_tot + comb_v[pl.ds(r * BINS + k * LANES, LANES)]
                c_tot = c_tot + comb_v[
                    pl.ds(NSUB * BINS + r * BINS + k * LANES, LANES)]
            num = s_tot + c_tot * blv
            out_v[pl.ds(k * LANES, LANES)] = num / jnp.maximum(c_tot, 1.0)
        pltpu.sync_copy(out_v, out_hbm)


_sc_pool = functools.partial(
    pl.kernel,
    out_type=jax.ShapeDtypeStruct((N_GRAPHS,), jnp.float32),
    mesh=plsc.VectorSubcoreMesh(core_axis_name="c", subcore_axis_name="s",
                                num_cores=1, num_subcores=NSUB),
    compiler_params=pltpu.CompilerParams(needs_layout_passes=False),
    scratch_types=[
        pltpu.VMEM((CHUNK,), jnp.float32),
        pltpu.VMEM((CHUNK,), jnp.int32),
        pltpu.VMEM((CHUNK,), jnp.int32),
        pltpu.VMEM((BINS,), jnp.float32),
        pltpu.VMEM((BINS,), jnp.float32),
        pltpu.VMEM_SHARED((NSUB * BINS,), jnp.float32),
        pltpu.VMEM_SHARED((NSUB * BINS,), jnp.float32),
        pltpu.VMEM((2 * NSUB * BINS,), jnp.float32),
        pltpu.VMEM((N_GRAPHS,), jnp.float32),
        pltpu.VMEM((LANES,), jnp.float32),
    ],
)(_sc_body)


def kernel(x, edge_index, edge_weight, batch, Wz0, Wz1, bz, Wr0, Wr1, br,
           Wh0, Wh1, bh, Wl, bl):
    del edge_index, edge_weight, Wr0, Wr1, br
    g = pl.pallas_call(
        _tc_body,
        out_shape=jax.ShapeDtypeStruct((N, 1), jnp.float32),
    )(x, Wz0, Wz1, bz.reshape(1, F_H), Wh0, Wh1, bh.reshape(1, F_H), Wl)
    g_pad = jnp.pad(g.reshape(N), (0, NPAD - N))
    b_pad = jnp.pad(batch, (0, NPAD - N), constant_values=PAD_ID)
    b_prev = jnp.concatenate(
        [jnp.full((1,), FORCE_ID, jnp.int32), b_pad[:-1]])
    b_prev = b_prev.at[::CHUNK].set(FORCE_ID)
    bl16 = jnp.broadcast_to(bl, (LANES,)).astype(jnp.float32)
    out = _sc_pool(g_pad, b_pad, b_prev, bl16)
    return out.reshape(N_GRAPHS, 1)


# speedup vs baseline: 1.0633x; 1.0633x over previous
import functools

import jax
import jax.numpy as jnp
from jax import lax
from jax.experimental import pallas as pl
from jax.experimental.pallas import tpu as pltpu
from jax.experimental.pallas import tpu_sc as plsc

N = 10000
F_IN = 128
F_H = 32
N_GRAPHS = 64
BLK = 2000

NSUB = 16
LANES = 16
NPAD = 10240
CHUNK = NPAD // NSUB
VREGS = CHUNK // LANES
BINS = 80
PAD_ID = 64
FORCE_ID = 66


def _tc_body(x_ref, wz0_ref, wz1_ref, bz_ref, wh0_ref, wh1_ref, bh_ref,
             wl_ref, g_ref):
    x = x_ref[...]
    w = jnp.concatenate(
        [wz0_ref[0:F_IN, :] + wz1_ref[0:F_IN, :],
         wh0_ref[0:F_IN, :] + wh1_ref[0:F_IN, :]], axis=1)
    p = jnp.dot(x, w, preferred_element_type=jnp.float32)
    z = jax.nn.sigmoid(p[:, 0:F_H] + bz_ref[...])
    t = jnp.tanh(p[:, F_H:2 * F_H] + bh_ref[...])
    hr = jnp.maximum((1.0 - z) * t, 0.0)
    g_ref[...] = jnp.dot(hr, wl_ref[...], preferred_element_type=jnp.float32)


def _gather16(v, idx):
    return lax.gather(
        v, idx[:, None],
        dimension_numbers=lax.GatherDimensionNumbers(
            offset_dims=(), collapsed_slice_dims=(0,), start_index_map=(0,)),
        slice_sizes=(1,), mode=lax.GatherScatterMode.PROMISE_IN_BOUNDS)


def _sc_body(g_hbm, b_hbm, bl_hbm, out_hbm,
             g_v, b_v, acc_s, acc_c, stage_s, stage_c, comb_v, out_v, bl_v):
    wid = lax.axis_index("s")
    base = wid * CHUNK
    pltpu.sync_copy(g_hbm.at[pl.ds(base, CHUNK)], g_v)
    pltpu.sync_copy(b_hbm.at[pl.ds(base, CHUNK)], b_v)

    zero16 = jnp.zeros((LANES,), jnp.float32)
    for k in range(BINS // LANES):
        acc_s[pl.ds(k * LANES, LANES)] = zero16
        acc_c[pl.ds(k * LANES, LANES)] = zero16

    lane = lax.iota(jnp.int32, LANES)
    shift_idx = jnp.maximum(lane - 1, 0)
    last_idx = jnp.full((LANES,), LANES - 1, jnp.int32)

    def step(j, carry):
        carry_sum, carry_id = carry
        v = g_v[pl.ds(j * LANES, LANES)]
        bb = b_v[pl.ds(j * LANES, LANES)]
        bp = jnp.where(lane == 0, carry_id, _gather16(bb, shift_idx))
        cs = plsc.cumsum(v)
        excl = carry_sum + cs - v
        mask = bb != bp
        plsc.addupdate_scatter(acc_s, [bb], -excl, mask=mask)
        plsc.addupdate_scatter(acc_s, [bp], excl, mask=mask)
        lpos = (j * LANES + lane).astype(jnp.float32)
        plsc.addupdate_scatter(acc_c, [bb], -lpos, mask=mask)
        plsc.addupdate_scatter(acc_c, [bp], lpos, mask=mask)
        return (carry_sum + _gather16(cs, last_idx), _gather16(bb, last_idx))
    carry_sum, _ = lax.fori_loop(
        0, VREGS, step,
        (zero16, jnp.full((LANES,), FORCE_ID, jnp.int32)))

    bb_last = b_v[pl.ds((VREGS - 1) * LANES, LANES)]
    last_lane = lane == (LANES - 1)
    plsc.addupdate_scatter(acc_s, [bb_last], carry_sum, mask=last_lane)
    plsc.addupdate_scatter(acc_c, [bb_last],
                           jnp.full((LANES,), float(CHUNK), jnp.float32),
                           mask=last_lane)

    pltpu.sync_copy(acc_s, stage_s.at[pl.ds(wid * BINS, BINS)])
    pltpu.sync_copy(acc_c, stage_c.at[pl.ds(wid * BINS, BINS)])
    plsc.subcore_barrier()

    @pl.when(wid == 0)
    def _():
        pltpu.sync_copy(bl_hbm, bl_v)
        pltpu.sync_copy(stage_s, comb_v.at[pl.ds(0, NSUB * BINS)])
        pltpu.sync_copy(stage_c, comb_v.at[pl.ds(NSUB * BINS, NSUB * BINS)])
        blv = bl_v[...]
        for k in range(N_GRAPHS // LANES):
            s_tot = zero16
            c_tot = zero16
            for r in range(NSUB):
                s_tot = s_tot + comb_v[pl.ds(r * BINS + k * LANES, LANES)]
                c_tot = c_tot + comb_v[
                    pl.ds(NSUB * BINS + r * BINS + k * LANES, LANES)]
            num = s_tot + c_tot * blv
            out_v[pl.ds(k * LANES, LANES)] = num / jnp.maximum(c_tot, 1.0)
        pltpu.sync_copy(out_v, out_hbm)


_sc_pool = functools.partial(
    pl.kernel,
    out_type=jax.ShapeDtypeStruct((N_GRAPHS,), jnp.float32),
    mesh=plsc.VectorSubcoreMesh(core_axis_name="c", subcore_axis_name="s",
                                num_cores=1, num_subcores=NSUB),
    compiler_params=pltpu.CompilerParams(needs_layout_passes=False),
    scratch_types=[
        pltpu.VMEM((CHUNK,), jnp.float32),
        pltpu.VMEM((CHUNK,), jnp.int32),
        pltpu.VMEM((BINS,), jnp.float32),
        pltpu.VMEM((BINS,), jnp.float32),
        pltpu.VMEM_SHARED((NSUB * BINS,), jnp.float32),
        pltpu.VMEM_SHARED((NSUB * BINS,), jnp.float32),
        pltpu.VMEM((2 * NSUB * BINS,), jnp.float32),
        pltpu.VMEM((N_GRAPHS,), jnp.float32),
        pltpu.VMEM((LANES,), jnp.float32),
    ],
)(_sc_body)


def kernel(x, edge_index, edge_weight, batch, Wz0, Wz1, bz, Wr0, Wr1, br,
           Wh0, Wh1, bh, Wl, bl):
    del edge_index, edge_weight, Wr0, Wr1, br
    g = pl.pallas_call(
        _tc_body,
        grid=(N // BLK,),
        in_specs=[
            pl.BlockSpec((BLK, F_IN), lambda i: (i, 0)),
            pl.BlockSpec((F_IN + F_H, F_H), lambda i: (0, 0)),
            pl.BlockSpec((F_IN + F_H, F_H), lambda i: (0, 0)),
            pl.BlockSpec((1, F_H), lambda i: (0, 0)),
            pl.BlockSpec((F_IN + F_H, F_H), lambda i: (0, 0)),
            pl.BlockSpec((F_IN + F_H, F_H), lambda i: (0, 0)),
            pl.BlockSpec((1, F_H), lambda i: (0, 0)),
            pl.BlockSpec((F_H, 1), lambda i: (0, 0)),
        ],
        out_specs=pl.BlockSpec((BLK, 1), lambda i: (i, 0)),
        out_shape=jax.ShapeDtypeStruct((N, 1), jnp.float32),
    )(x, Wz0, Wz1, bz.reshape(1, F_H), Wh0, Wh1, bh.reshape(1, F_H), Wl)
    g_pad = jnp.pad(g.reshape(N), (0, NPAD - N))
    b_pad = jnp.pad(batch, (0, NPAD - N), constant_values=PAD_ID)
    bl16 = jnp.broadcast_to(bl, (LANES,)).astype(jnp.float32)
    out = _sc_pool(g_pad, b_pad, bl16)
    return out.reshape(N_GRAPHS, 1)
